# baseline (speedup 1.0000x reference)
import functools

import jax
import jax.numpy as jnp
from jax import lax
from jax.experimental import pallas as pl
from jax.experimental.pallas import tpu as pltpu
from jax.experimental.pallas import tpu_sc as plsc

VOCAB_SIZE = 49408
HIDDEN = 512
MAX_POS = 16
BATCH = 16384
SEQ_LEN = 16

TOTAL = BATCH * SEQ_LEN
NUM_WORKERS = 32
PER_WORKER = TOTAL // NUM_WORKERS
CHUNK = 32
NCHUNKS = PER_WORKER // CHUNK
NBUF = 4

_mesh = plsc.VectorSubcoreMesh(core_axis_name="c", subcore_axis_name="s")


@functools.partial(
    pl.kernel,
    out_type=jax.ShapeDtypeStruct((TOTAL, HIDDEN), jnp.float32),
    mesh=_mesh,
    scratch_types=[
        pltpu.VMEM((NCHUNKS, CHUNK), jnp.int32),
        pltpu.VMEM((MAX_POS, HIDDEN), jnp.float32),
    ]
    + [pltpu.VMEM((CHUNK, HIDDEN), jnp.float32) for _ in range(NBUF)]
    + [pltpu.SemaphoreType.DMA for _ in range(2 * NBUF)],
)
def _gather_add(table_hbm, idx_hbm, pos_hbm, out_hbm, idx_v, pos_v, *bufs):
    rows = bufs[:NBUF]
    gsem = bufs[NBUF : 2 * NBUF]
    ssem = bufs[2 * NBUF :]

    wid = lax.axis_index("s") * 2 + lax.axis_index("c")
    base = wid * PER_WORKER

    pltpu.sync_copy(idx_hbm.at[wid], idx_v)
    pltpu.sync_copy(pos_hbm, pos_v)

    def gather_copy(c, b):
        return pltpu.make_async_copy(table_hbm.at[idx_v.at[c]], rows[b], gsem[b])

    def store_copy(c, b):
        return pltpu.make_async_copy(
            rows[b], out_hbm.at[pl.ds(base + c * CHUNK, CHUNK)], ssem[b]
        )

    def add_pos(b):
        @plsc.parallel_loop(0, MAX_POS, step=1)
        def body(p):
            for j in range(HIDDEN // 16):
                sl = pl.ds(j * 16, 16)
                x = pos_v[p, sl]
                plsc.addupdate(rows[b].at[p, sl], x)
                plsc.addupdate(rows[b].at[p + MAX_POS, sl], x)

    def step(c, b, first, last):
        gather_copy(c, b).wait()
        add_pos(b)
        store_copy(c, b).start()
        pb = (b - 1) % NBUF
        if not first:
            store_copy(c - 1, pb).wait()
        if not last:
            gather_copy(c + NBUF - 1, pb).start()

    for b in range(NBUF - 1):
        gather_copy(b, b).start()

    for b in range(NBUF):
        step(b, b, first=(b == 0), last=False)

    def outer(i, carry):
        c0 = i * NBUF
        for b in range(NBUF):
            step(c0 + b, b, first=False, last=False)
        return carry

    lax.fori_loop(1, NCHUNKS // NBUF - 1, outer, 0, unroll=2)

    for b in range(NBUF):
        c = NCHUNKS - NBUF + b
        step(c, b, first=False, last=(b != 0))

    store_copy(NCHUNKS - 1, NBUF - 1).wait()


def kernel(input_ids, token_embedding, position_embedding):
    idx = input_ids.astype(jnp.int32).reshape(NUM_WORKERS, NCHUNKS, CHUNK)
    out = _gather_add(token_embedding, idx, position_embedding)
    return out.reshape(BATCH, SEQ_LEN, HIDDEN)


# speedup vs baseline: 1.0246x; 1.0246x over previous
import functools

import jax
import jax.numpy as jnp
from jax import lax
from jax.experimental import pallas as pl
from jax.experimental.pallas import tpu as pltpu
from jax.experimental.pallas import tpu_sc as plsc

VOCAB_SIZE = 49408
HIDDEN = 512
MAX_POS = 16
BATCH = 16384
SEQ_LEN = 16

TOTAL = BATCH * SEQ_LEN
NUM_WORKERS = 32
PER_WORKER = TOTAL // NUM_WORKERS
CHUNK = 64
NCHUNKS = PER_WORKER // CHUNK
NBUF = 3
NMAIN = NCHUNKS - (NCHUNKS % NBUF or NBUF)
NTAIL = NCHUNKS - NMAIN

_mesh = plsc.VectorSubcoreMesh(core_axis_name="c", subcore_axis_name="s")


@functools.partial(
    pl.kernel,
    out_type=jax.ShapeDtypeStruct((TOTAL, HIDDEN), jnp.float32),
    mesh=_mesh,
    scratch_types=[
        pltpu.VMEM((NCHUNKS, CHUNK), jnp.int32),
        pltpu.VMEM((MAX_POS, HIDDEN), jnp.float32),
    ]
    + [pltpu.VMEM((CHUNK, HIDDEN), jnp.float32) for _ in range(NBUF)]
    + [pltpu.SemaphoreType.DMA for _ in range(2 * NBUF)],
)
def _gather_add(table_hbm, idx_hbm, pos_hbm, out_hbm, idx_v, pos_v, *bufs):
    rows = bufs[:NBUF]
    gsem = bufs[NBUF : 2 * NBUF]
    ssem = bufs[2 * NBUF :]

    wid = lax.axis_index("s") * 2 + lax.axis_index("c")
    base = wid * PER_WORKER

    pltpu.sync_copy(idx_hbm.at[wid], idx_v)
    pltpu.sync_copy(pos_hbm, pos_v)

    def gather_copy(c, b):
        return pltpu.make_async_copy(table_hbm.at[idx_v.at[c]], rows[b], gsem[b])

    def store_copy(c, b):
        return pltpu.make_async_copy(
            rows[b], out_hbm.at[pl.ds(base + c * CHUNK, CHUNK)], ssem[b]
        )

    def add_pos(b):
        @plsc.parallel_loop(0, MAX_POS, step=1)
        def body(p):
            for j in range(HIDDEN // 16):
                sl = pl.ds(j * 16, 16)
                x = pos_v[p, sl]
                for rep in range(CHUNK // MAX_POS):
                    plsc.addupdate(rows[b].at[p + rep * MAX_POS, sl], x)

    def step(c, b, first, last):
        gather_copy(c, b).wait()
        add_pos(b)
        store_copy(c, b).start()
        pb = (b - 1) % NBUF
        if not first:
            store_copy(c - 1, pb).wait()
        if not last:
            gather_copy(c + NBUF - 1, pb).start()

    for b in range(NBUF - 1):
        gather_copy(b, b).start()

    for b in range(NBUF):
        step(b, b, first=(b == 0), last=False)

    def outer(i, carry):
        c0 = i * NBUF
        for b in range(NBUF):
            step(c0 + b, b, first=False, last=False)
        return carry

    lax.fori_loop(1, NMAIN // NBUF, outer, 0, unroll=False)

    for c in range(NMAIN, NCHUNKS):
        step(c, c % NBUF, first=False, last=(c > NCHUNKS - NBUF))

    store_copy(NCHUNKS - 1, (NCHUNKS - 1) % NBUF).wait()


def kernel(input_ids, token_embedding, position_embedding):
    idx = input_ids.astype(jnp.int32).reshape(NUM_WORKERS, NCHUNKS, CHUNK)
    out = _gather_add(token_embedding, idx, position_embedding)
    return out.reshape(BATCH, SEQ_LEN, HIDDEN)
